# baseline (speedup 1.0000x reference)
import functools

import jax
import jax.numpy as jnp
from jax import lax
from jax.experimental import pallas as pl

N = 10000
E = 160000
D = 256
L = 3

_ROWS = 2000


def _layer_body(m_ref, nd_ref, w_ref, b_ref, o_ref):
    m = m_ref[...]
    y = jnp.dot(m, w_ref[...], preferred_element_type=jnp.float32,
                precision=lax.Precision.HIGHEST)
    y = y * nd_ref[...] + b_ref[...]
    o_ref[...] = jnp.maximum(y, 0.0)


def _layer_tc(m, nd, w, b):
    return pl.pallas_call(
        _layer_body,
        grid=(N // _ROWS,),
        in_specs=[
            pl.BlockSpec((_ROWS, D), lambda i: (i, 0)),
            pl.BlockSpec((_ROWS, 1), lambda i: (i, 0)),
            pl.BlockSpec((D, D), lambda i: (0, 0)),
            pl.BlockSpec((1, D), lambda i: (0, 0)),
        ],
        out_specs=pl.BlockSpec((_ROWS, D), lambda i: (i, 0)),
        out_shape=jax.ShapeDtypeStruct((N, D), jnp.float32),
    )(m, nd, w, b)


def _pool_body(h_ref, wl_ref, bl_ref, o_ref, acc_ref):
    i = pl.program_id(0)

    @pl.when(i == 0)
    def _():
        acc_ref[...] = jnp.zeros_like(acc_ref)

    acc_ref[...] += jnp.sum(h_ref[...], axis=0, keepdims=True)

    @pl.when(i == pl.num_programs(0) - 1)
    def _():
        pooled = acc_ref[...] * (1.0 / N)
        o_ref[...] = jnp.dot(pooled, wl_ref[...],
                             preferred_element_type=jnp.float32,
                             precision=lax.Precision.HIGHEST) + bl_ref[...]


def _pool_tc(h, wl, bl):
    return pl.pallas_call(
        _pool_body,
        grid=(N // _ROWS,),
        in_specs=[
            pl.BlockSpec((_ROWS, D), lambda i: (i, 0)),
            pl.BlockSpec((D, D), lambda i: (0, 0)),
            pl.BlockSpec((1, D), lambda i: (0, 0)),
        ],
        out_specs=pl.BlockSpec((1, D), lambda i: (0, 0)),
        out_shape=jax.ShapeDtypeStruct((1, D), jnp.float32),
        scratch_shapes=[pltpu_vmem((1, D), jnp.float32)],
    )(h, wl, bl)


def pltpu_vmem(shape, dtype):
    from jax.experimental.pallas import tpu as pltpu
    return pltpu.VMEM(shape, dtype)


@jax.jit
def _run(h, src, dst, Ws, bs, Wl, bl):
    deg_out = jnp.maximum(jnp.bincount(src, length=N), 1).astype(jnp.float32)
    deg_in = jnp.maximum(jnp.bincount(dst, length=N), 1).astype(jnp.float32)
    ns = (deg_out ** -0.5).reshape(N, 1)
    nd = (deg_in ** -0.5).reshape(N, 1)
    for i in range(L):
        x = h * ns
        m = jax.ops.segment_sum(x[src], dst, num_segments=N)
        h = _layer_tc(m, nd, Ws[i], bs[i].reshape(1, D))
    global_feat = _pool_tc(h, Wl, bl.reshape(1, D))
    return h, global_feat


def kernel(h, edge_index, he, Ws, bs, Wl, bl):
    src = edge_index[0].astype(jnp.int32)
    dst = edge_index[1].astype(jnp.int32)
    return _run(h, src, dst, Ws, bs, Wl, bl)


# speedup vs baseline: 2.8243x; 2.8243x over previous
import functools

import jax
import jax.numpy as jnp
from jax import lax
from jax.experimental import pallas as pl
from jax.experimental.pallas import tpu as pltpu
from jax.experimental.pallas import tpu_sc as plsc

N = 10000
E = 160000
D = 256
L = 3
H = 128

NT = 16
NCORES = 2
CHUNK = 128
G = 79
E_PAD = NT * G * CHUNK
TRASH = N

ACC_R = NT * 632
DEG_R = NT * 632

_ROWS = 2000

_MESH = plsc.VectorSubcoreMesh(core_axis_name="c", subcore_axis_name="s")


@functools.partial(
    pl.kernel,
    out_type=jax.ShapeDtypeStruct((2 * DEG_R,), jnp.float32),
    mesh=_MESH,
    scratch_types=[
        pltpu.VMEM((CHUNK,), jnp.int32),
        pltpu.VMEM((CHUNK,), jnp.float32),
        pltpu.VMEM((632,), jnp.float32),
        pltpu.VMEM_SHARED((DEG_R,), jnp.float32),
    ],
)
def _deg_sc(degidx_hbm, ones_hbm, zeros_hbm, out_hbm, idx_v, ones_v, stage_v,
            acc):
    c = lax.axis_index("c")
    t = lax.axis_index("s")
    pltpu.sync_copy(zeros_hbm.at[pl.ds(0, 632)], stage_v)
    pltpu.sync_copy(stage_v, acc.at[pl.ds(t * 632, 632)])
    pltpu.sync_copy(ones_hbm, ones_v)
    plsc.subcore_barrier()

    def chunk(g, carry):
        b = t * (G * CHUNK) + g * CHUNK
        pltpu.sync_copy(degidx_hbm.at[pl.ds(c * E_PAD + b, CHUNK)], idx_v)
        pltpu.sync_copy(ones_v, acc.at[idx_v], add=True)
        return carry

    lax.fori_loop(0, G, chunk, 0)
    plsc.subcore_barrier()
    pltpu.sync_copy(acc.at[pl.ds(t * 632, 632)], stage_v)
    pltpu.sync_copy(stage_v, out_hbm.at[pl.ds(c * DEG_R + t * 632, 632)])


@functools.partial(
    pl.kernel,
    out_type=jax.ShapeDtypeStruct((2 * N, H), jnp.float32),
    mesh=_MESH,
    scratch_types=[
        pltpu.VMEM((CHUNK,), jnp.int32),
        pltpu.VMEM((CHUNK,), jnp.int32),
        pltpu.VMEM((CHUNK, H), jnp.float32),
        pltpu.VMEM_SHARED((ACC_R, H), jnp.float32),
        pltpu.SemaphoreType.DMA,
    ],
)
def _agg_sc(xs_hbm, src2_hbm, dst_hbm, zeros_hbm, out_hbm,
            sidx_v, didx_v, rows_v, acc, sem):
    c = lax.axis_index("c")
    t = lax.axis_index("s")
    _SPANS = ((0, CHUNK), (1, CHUNK), (2, CHUNK), (3, CHUNK), (4, 632 - 4 * CHUNK))
    pltpu.sync_copy(zeros_hbm, rows_v)
    for k, sz in _SPANS:
        pltpu.sync_copy(rows_v.at[pl.ds(0, sz)],
                        acc.at[pl.ds(t * 632 + k * CHUNK, sz)])
    plsc.subcore_barrier()

    def chunk(g, carry):
        b = t * (G * CHUNK) + g * CHUNK
        pltpu.sync_copy(src2_hbm.at[pl.ds(c * E_PAD + b, CHUNK)], sidx_v)
        pltpu.sync_copy(dst_hbm.at[pl.ds(b, CHUNK)], didx_v)
        pltpu.async_copy(xs_hbm.at[sidx_v], rows_v, sem).wait()
        pltpu.sync_copy(rows_v, acc.at[didx_v], add=True)
        return carry

    lax.fori_loop(0, G, chunk, 0)
    plsc.subcore_barrier()

    @pl.when(t < NT - 1)
    def _():
        for k, sz in ((0, CHUNK), (1, CHUNK), (2, CHUNK), (3, CHUNK),
                      (4, 632 - 4 * CHUNK)):
            pltpu.sync_copy(acc.at[pl.ds(t * 632 + k * CHUNK, sz)],
                            rows_v.at[pl.ds(0, sz)])
            pltpu.sync_copy(rows_v.at[pl.ds(0, sz)],
                            out_hbm.at[pl.ds(c * N + t * 632 + k * CHUNK, sz)])

    @pl.when(t == NT - 1)
    def _():
        for k, sz in ((0, CHUNK), (1, CHUNK), (2, CHUNK), (3, CHUNK),
                      (4, 520 - 4 * CHUNK)):
            pltpu.sync_copy(acc.at[pl.ds(9480 + k * CHUNK, sz)],
                            rows_v.at[pl.ds(0, sz)])
            pltpu.sync_copy(rows_v.at[pl.ds(0, sz)],
                            out_hbm.at[pl.ds(c * N + 9480 + k * CHUNK, sz)])


def _norm(deg):
    return jax.lax.rsqrt(jnp.maximum(deg, 1.0))


def _prescale_body(h_ref, dego_ref, o_ref):
    o_ref[...] = h_ref[...] * _norm(dego_ref[...])


def _prescale_tc(h, deg_out):
    return pl.pallas_call(
        _prescale_body,
        grid=(N // _ROWS, 2),
        in_specs=[
            pl.BlockSpec((_ROWS, H), lambda i, j: (i, j)),
            pl.BlockSpec((_ROWS, 1), lambda i, j: (i, 0)),
        ],
        out_specs=pl.BlockSpec((_ROWS, H), lambda i, j: (j * (N // _ROWS) + i, 0)),
        out_shape=jax.ShapeDtypeStruct((2 * N, H), jnp.float32),
    )(h, deg_out)


def _layer_compute(mlo_ref, mhi_ref, degi_ref, wlo_ref, whi_ref, b_ref):
    y = jnp.dot(mlo_ref[...], wlo_ref[...], preferred_element_type=jnp.float32,
                precision=lax.Precision.HIGHEST)
    y += jnp.dot(mhi_ref[...], whi_ref[...], preferred_element_type=jnp.float32,
                 precision=lax.Precision.HIGHEST)
    y = y * _norm(degi_ref[...]) + b_ref[...]
    return jnp.maximum(y, 0.0)


def _layer_mid_body(mlo_ref, mhi_ref, degi_ref, dego_ref, wlo_ref, whi_ref,
                    b_ref, o_ref):
    y = _layer_compute(mlo_ref, mhi_ref, degi_ref, wlo_ref, whi_ref, b_ref)
    o_ref[...] = y * _norm(dego_ref[...])


def _layer_mid_tc(m2, deg_in, deg_out, w, b):
    nb = N // _ROWS
    return pl.pallas_call(
        _layer_mid_body,
        grid=(nb, 2),
        in_specs=[
            pl.BlockSpec((_ROWS, H), lambda i, j: (i, 0)),
            pl.BlockSpec((_ROWS, H), lambda i, j: (nb + i, 0)),
            pl.BlockSpec((_ROWS, 1), lambda i, j: (i, 0)),
            pl.BlockSpec((_ROWS, 1), lambda i, j: (i, 0)),
            pl.BlockSpec((H, H), lambda i, j: (0, j)),
            pl.BlockSpec((H, H), lambda i, j: (1, j)),
            pl.BlockSpec((1, H), lambda i, j: (0, j)),
        ],
        out_specs=pl.BlockSpec((_ROWS, H), lambda i, j: (j * nb + i, 0)),
        out_shape=jax.ShapeDtypeStruct((2 * N, H), jnp.float32),
    )(m2, m2, deg_in, deg_out, w, w, b)


def _layer_fin_body(mlo_ref, mhi_ref, degi_ref, wlo_ref, whi_ref, b_ref, o_ref):
    o_ref[...] = _layer_compute(mlo_ref, mhi_ref, degi_ref, wlo_ref, whi_ref,
                                b_ref)


def _layer_fin_tc(m2, deg_in, w, b):
    nb = N // _ROWS
    return pl.pallas_call(
        _layer_fin_body,
        grid=(nb, 2),
        in_specs=[
            pl.BlockSpec((_ROWS, H), lambda i, j: (i, 0)),
            pl.BlockSpec((_ROWS, H), lambda i, j: (nb + i, 0)),
            pl.BlockSpec((_ROWS, 1), lambda i, j: (i, 0)),
            pl.BlockSpec((H, H), lambda i, j: (0, j)),
            pl.BlockSpec((H, H), lambda i, j: (1, j)),
            pl.BlockSpec((1, H), lambda i, j: (0, j)),
        ],
        out_specs=pl.BlockSpec((_ROWS, H), lambda i, j: (i, j)),
        out_shape=jax.ShapeDtypeStruct((N, D), jnp.float32),
    )(m2, m2, deg_in, w, w, b)


def _pool_body(h_ref, wl_ref, bl_ref, o_ref, acc_ref):
    i = pl.program_id(0)

    @pl.when(i == 0)
    def _():
        acc_ref[...] = jnp.zeros_like(acc_ref)

    acc_ref[...] += jnp.sum(h_ref[...], axis=0, keepdims=True)

    @pl.when(i == pl.num_programs(0) - 1)
    def _():
        pooled = acc_ref[...] * (1.0 / N)
        o_ref[...] = jnp.dot(pooled, wl_ref[...],
                             preferred_element_type=jnp.float32,
                             precision=lax.Precision.HIGHEST) + bl_ref[...]


def _pool_tc(h, wl, bl):
    return pl.pallas_call(
        _pool_body,
        grid=(N // _ROWS,),
        in_specs=[
            pl.BlockSpec((_ROWS, D), lambda i: (i, 0)),
            pl.BlockSpec((D, D), lambda i: (0, 0)),
            pl.BlockSpec((1, D), lambda i: (0, 0)),
        ],
        out_specs=pl.BlockSpec((1, D), lambda i: (0, 0)),
        out_shape=jax.ShapeDtypeStruct((1, D), jnp.float32),
        scratch_shapes=[pltpu.VMEM((1, D), jnp.float32)],
    )(h, wl, bl)


@jax.jit
def _run(h, src, dst, Ws, bs, Wl, bl):
    pad = E_PAD - E
    src_pad = jnp.concatenate([src, jnp.zeros((pad,), jnp.int32)])
    src2 = jnp.concatenate([src_pad, src_pad + N])
    dst_pad = jnp.concatenate([dst, jnp.full((pad,), TRASH, jnp.int32)])
    degidx = jnp.concatenate([
        jnp.concatenate([src, jnp.full((pad,), TRASH, jnp.int32)]),
        dst_pad,
    ])
    zeros_arr = jnp.zeros((CHUNK, H), jnp.float32)
    ones_arr = jnp.ones((CHUNK,), jnp.float32)

    degs = _deg_sc(degidx, ones_arr, zeros_arr.reshape(-1))
    deg_out = degs[0:N].reshape(N, 1)
    deg_in = degs[DEG_R:DEG_R + N].reshape(N, 1)

    xs = _prescale_tc(h, deg_out)
    for i in range(L):
        m2 = _agg_sc(xs, src2, dst_pad, zeros_arr)
        if i < L - 1:
            xs = _layer_mid_tc(m2, deg_in, deg_out, Ws[i], bs[i].reshape(1, D))
        else:
            local_feat = _layer_fin_tc(m2, deg_in, Ws[i], bs[i].reshape(1, D))
    global_feat = _pool_tc(local_feat, Wl, bl.reshape(1, D))
    return local_feat, global_feat


def kernel(h, edge_index, he, Ws, bs, Wl, bl):
    src = edge_index[0].astype(jnp.int32)
    dst = edge_index[1].astype(jnp.int32)
    return _run(h, src, dst, Ws, bs, Wl, bl)


# speedup vs baseline: 3.1237x; 1.1060x over previous
import functools

import jax
import jax.numpy as jnp
from jax import lax
from jax.experimental import pallas as pl
from jax.experimental.pallas import tpu as pltpu
from jax.experimental.pallas import tpu_sc as plsc

N = 10000
E = 160000
D = 256
L = 3
H = 128

NT = 16
NCORES = 2
CHUNK = 128
G = 80
GP = G // 2
E_PAD = NT * G * CHUNK
TRASH = N

ACC_R = NT * 632
DEG_R = NT * 632

_ROWS = 2000

_MESH = plsc.VectorSubcoreMesh(core_axis_name="c", subcore_axis_name="s")


@functools.partial(
    pl.kernel,
    out_type=jax.ShapeDtypeStruct((2 * DEG_R,), jnp.float32),
    mesh=_MESH,
    scratch_types=[
        pltpu.VMEM((CHUNK,), jnp.int32),
        pltpu.VMEM((CHUNK,), jnp.float32),
        pltpu.VMEM((632,), jnp.float32),
        pltpu.VMEM_SHARED((DEG_R,), jnp.float32),
    ],
)
def _deg_sc(degidx_hbm, ones_hbm, zeros_hbm, out_hbm, idx_v, ones_v, stage_v,
            acc):
    c = lax.axis_index("c")
    t = lax.axis_index("s")
    pltpu.sync_copy(zeros_hbm.at[pl.ds(0, 632)], stage_v)
    pltpu.sync_copy(stage_v, acc.at[pl.ds(t * 632, 632)])
    pltpu.sync_copy(ones_hbm, ones_v)
    plsc.subcore_barrier()

    def chunk(g, carry):
        b = t * (G * CHUNK) + g * CHUNK
        pltpu.sync_copy(degidx_hbm.at[pl.ds(c * E_PAD + b, CHUNK)], idx_v)
        pltpu.sync_copy(ones_v, acc.at[idx_v], add=True)
        return carry

    lax.fori_loop(0, G, chunk, 0)
    plsc.subcore_barrier()
    pltpu.sync_copy(acc.at[pl.ds(t * 632, 632)], stage_v)
    pltpu.sync_copy(stage_v, out_hbm.at[pl.ds(c * DEG_R + t * 632, 632)])


@functools.partial(
    pl.kernel,
    out_type=jax.ShapeDtypeStruct((2 * N, H), jnp.float32),
    mesh=_MESH,
    scratch_types=[
        pltpu.VMEM((GP, 2, CHUNK), jnp.int32),
        pltpu.VMEM((2, CHUNK, H), jnp.float32),
        pltpu.VMEM_SHARED((ACC_R, H), jnp.float32),
        pltpu.SemaphoreType.DMA,
        pltpu.SemaphoreType.DMA,
    ],
)
def _agg_sc(xs_hbm, comb_hbm, zeros_hbm, out_hbm,
            idxb, rows_v, acc, sem0, sem1):
    c = lax.axis_index("c")
    t = lax.axis_index("s")
    sems = (sem0, sem1)

    stage = rows_v.at[0]
    pltpu.sync_copy(zeros_hbm, stage)
    for k, sz in ((0, CHUNK), (1, CHUNK), (2, CHUNK), (3, CHUNK),
                  (4, 632 - 4 * CHUNK)):
        pltpu.sync_copy(stage.at[pl.ds(0, sz)],
                        acc.at[pl.ds(t * 632 + k * CHUNK, sz)])
    plsc.subcore_barrier()

    base = (c * NT + t) * G

    def issue_gather(j, b):
        pltpu.async_copy(xs_hbm.at[idxb.at[j, 0]], rows_v.at[b], sems[b])

    def consume(j, b):
        pltpu.make_async_copy(xs_hbm.at[idxb.at[j, 0]], rows_v.at[b],
                              sems[b]).wait()
        pltpu.sync_copy(rows_v.at[b], acc.at[idxb.at[j, 1]], add=True)

    for p in range(2):
        pltpu.sync_copy(comb_hbm.at[pl.ds(base + p * GP, GP)], idxb)
        issue_gather(0, 0)

        def body(i, carry):
            j = 2 * i + 1
            issue_gather(j, 1)
            consume(j - 1, 0)
            issue_gather(j + 1, 0)
            consume(j, 1)
            return carry

        lax.fori_loop(0, GP // 2 - 1, body, 0)
        j = GP - 1
        issue_gather(j, 1)
        consume(j - 1, 0)
        consume(j, 1)

    plsc.subcore_barrier()

    @pl.when(t < NT - 1)
    def _():
        for k, sz in ((0, CHUNK), (1, CHUNK), (2, CHUNK), (3, CHUNK),
                      (4, 632 - 4 * CHUNK)):
            pltpu.sync_copy(acc.at[pl.ds(t * 632 + k * CHUNK, sz)],
                            stage.at[pl.ds(0, sz)])
            pltpu.sync_copy(stage.at[pl.ds(0, sz)],
                            out_hbm.at[pl.ds(c * N + t * 632 + k * CHUNK, sz)])

    @pl.when(t == NT - 1)
    def _():
        for k, sz in ((0, CHUNK), (1, CHUNK), (2, CHUNK), (3, CHUNK),
                      (4, 520 - 4 * CHUNK)):
            pltpu.sync_copy(acc.at[pl.ds(9480 + k * CHUNK, sz)],
                            stage.at[pl.ds(0, sz)])
            pltpu.sync_copy(stage.at[pl.ds(0, sz)],
                            out_hbm.at[pl.ds(c * N + 9480 + k * CHUNK, sz)])


def _norm(deg):
    return jax.lax.rsqrt(jnp.maximum(deg, 1.0))


def _prescale_body(h_ref, dego_ref, o_ref):
    o_ref[...] = h_ref[...] * _norm(dego_ref[...])


def _prescale_tc(h, deg_out):
    return pl.pallas_call(
        _prescale_body,
        grid=(N // _ROWS, 2),
        in_specs=[
            pl.BlockSpec((_ROWS, H), lambda i, j: (i, j)),
            pl.BlockSpec((_ROWS, 1), lambda i, j: (i, 0)),
        ],
        out_specs=pl.BlockSpec((_ROWS, H), lambda i, j: (j * (N // _ROWS) + i, 0)),
        out_shape=jax.ShapeDtypeStruct((2 * N, H), jnp.float32),
    )(h, deg_out)


def _layer_compute(mlo_ref, mhi_ref, degi_ref, wlo_ref, whi_ref, b_ref):
    y = jnp.dot(mlo_ref[...], wlo_ref[...], preferred_element_type=jnp.float32,
                precision=lax.Precision.HIGHEST)
    y += jnp.dot(mhi_ref[...], whi_ref[...], preferred_element_type=jnp.float32,
                 precision=lax.Precision.HIGHEST)
    y = y * _norm(degi_ref[...]) + b_ref[...]
    return jnp.maximum(y, 0.0)


def _layer_mid_body(mlo_ref, mhi_ref, degi_ref, dego_ref, wlo_ref, whi_ref,
                    b_ref, o_ref):
    y = _layer_compute(mlo_ref, mhi_ref, degi_ref, wlo_ref, whi_ref, b_ref)
    o_ref[...] = y * _norm(dego_ref[...])


def _layer_mid_tc(m2, deg_in, deg_out, w, b):
    nb = N // _ROWS
    return pl.pallas_call(
        _layer_mid_body,
        grid=(nb, 2),
        in_specs=[
            pl.BlockSpec((_ROWS, H), lambda i, j: (i, 0)),
            pl.BlockSpec((_ROWS, H), lambda i, j: (nb + i, 0)),
            pl.BlockSpec((_ROWS, 1), lambda i, j: (i, 0)),
            pl.BlockSpec((_ROWS, 1), lambda i, j: (i, 0)),
            pl.BlockSpec((H, H), lambda i, j: (0, j)),
            pl.BlockSpec((H, H), lambda i, j: (1, j)),
            pl.BlockSpec((1, H), lambda i, j: (0, j)),
        ],
        out_specs=pl.BlockSpec((_ROWS, H), lambda i, j: (j * nb + i, 0)),
        out_shape=jax.ShapeDtypeStruct((2 * N, H), jnp.float32),
    )(m2, m2, deg_in, deg_out, w, w, b)


def _layer_fin_body(mlo_ref, mhi_ref, degi_ref, wlo_ref, whi_ref, b_ref, o_ref):
    o_ref[...] = _layer_compute(mlo_ref, mhi_ref, degi_ref, wlo_ref, whi_ref,
                                b_ref)


def _layer_fin_tc(m2, deg_in, w, b):
    nb = N // _ROWS
    return pl.pallas_call(
        _layer_fin_body,
        grid=(nb, 2),
        in_specs=[
            pl.BlockSpec((_ROWS, H), lambda i, j: (i, 0)),
            pl.BlockSpec((_ROWS, H), lambda i, j: (nb + i, 0)),
            pl.BlockSpec((_ROWS, 1), lambda i, j: (i, 0)),
            pl.BlockSpec((H, H), lambda i, j: (0, j)),
            pl.BlockSpec((H, H), lambda i, j: (1, j)),
            pl.BlockSpec((1, H), lambda i, j: (0, j)),
        ],
        out_specs=pl.BlockSpec((_ROWS, H), lambda i, j: (i, j)),
        out_shape=jax.ShapeDtypeStruct((N, D), jnp.float32),
    )(m2, m2, deg_in, w, w, b)


def _pool_body(h_ref, wl_ref, bl_ref, o_ref, acc_ref):
    i = pl.program_id(0)

    @pl.when(i == 0)
    def _():
        acc_ref[...] = jnp.zeros_like(acc_ref)

    acc_ref[...] += jnp.sum(h_ref[...], axis=0, keepdims=True)

    @pl.when(i == pl.num_programs(0) - 1)
    def _():
        pooled = acc_ref[...] * (1.0 / N)
        o_ref[...] = jnp.dot(pooled, wl_ref[...],
                             preferred_element_type=jnp.float32,
                             precision=lax.Precision.HIGHEST) + bl_ref[...]


def _pool_tc(h, wl, bl):
    return pl.pallas_call(
        _pool_body,
        grid=(N // _ROWS,),
        in_specs=[
            pl.BlockSpec((_ROWS, D), lambda i: (i, 0)),
            pl.BlockSpec((D, D), lambda i: (0, 0)),
            pl.BlockSpec((1, D), lambda i: (0, 0)),
        ],
        out_specs=pl.BlockSpec((1, D), lambda i: (0, 0)),
        out_shape=jax.ShapeDtypeStruct((1, D), jnp.float32),
        scratch_shapes=[pltpu.VMEM((1, D), jnp.float32)],
    )(h, wl, bl)


@jax.jit
def _run(h, src, dst, Ws, bs, Wl, bl):
    pad = E_PAD - E
    src_pad = jnp.concatenate([src, jnp.zeros((pad,), jnp.int32)])
    dst_pad = jnp.concatenate([dst, jnp.full((pad,), TRASH, jnp.int32)])
    s3 = src_pad.reshape(NT, G, CHUNK)
    d3 = dst_pad.reshape(NT, G, CHUNK)
    comb = jnp.stack([
        jnp.stack([s3, d3], axis=2),
        jnp.stack([s3 + N, d3], axis=2),
    ]).reshape(2 * NT * G, 2, CHUNK)
    degidx = jnp.concatenate([
        jnp.concatenate([src, jnp.full((pad,), TRASH, jnp.int32)]),
        dst_pad,
    ])
    zeros_arr = jnp.zeros((CHUNK, H), jnp.float32)
    ones_arr = jnp.ones((CHUNK,), jnp.float32)

    degs = _deg_sc(degidx, ones_arr, zeros_arr.reshape(-1))
    deg_out = degs[0:N].reshape(N, 1)
    deg_in = degs[DEG_R:DEG_R + N].reshape(N, 1)

    xs = _prescale_tc(h, deg_out)
    for i in range(L):
        m2 = _agg_sc(xs, comb, zeros_arr)
        if i < L - 1:
            xs = _layer_mid_tc(m2, deg_in, deg_out, Ws[i], bs[i].reshape(1, D))
        else:
            local_feat = _layer_fin_tc(m2, deg_in, Ws[i], bs[i].reshape(1, D))
    global_feat = _pool_tc(local_feat, Wl, bl.reshape(1, D))
    return local_feat, global_feat


def kernel(h, edge_index, he, Ws, bs, Wl, bl):
    src = edge_index[0].astype(jnp.int32)
    dst = edge_index[1].astype(jnp.int32)
    return _run(h, src, dst, Ws, bs, Wl, bl)
